# Initial kernel scaffold; baseline (speedup 1.0000x reference)
import jax
import jax.numpy as jnp
from jax.experimental import pallas as pl


def kernel(x, batch, W, b):
    raise NotImplementedError("write your pallas kernel here")


# speedup vs baseline: 1.6998x; 1.6998x over previous
import jax
import jax.numpy as jnp
from jax import lax
from jax.experimental import pallas as pl
from jax.experimental.pallas import tpu as pltpu
from jax.experimental.pallas import tpu_sc as plsc

N_NODES = 100000
C = 128
G = 1024
NC = 2
NS = 16
NW = NC * NS
SEG_PER_W = G // NW
K = 512
NCH = C // 16


def _pool_body(x_hbm, batch_hbm, starts_hbm, out_hbm, sbuf, ibuf, xbuf, acc):
    wid = lax.axis_index("s") * NC + lax.axis_index("c")
    g0 = wid * SEG_PER_W
    g1 = g0 + SEG_PER_W

    pltpu.sync_copy(starts_hbm, sbuf)
    bounds = sbuf[pl.ds(wid, 16)]
    lo = bounds[0]
    hi = bounds[1]
    base0 = (lo // 8) * 8
    nblk = (hi - base0 + (K - 1)) // K

    zero = jnp.zeros((16,), jnp.float32)
    for s in range(SEG_PER_W):
        for c in range(NCH):
            acc[s, pl.ds(c * 16, 16)] = zero

    def blk_body(blk, carry):
        base = base0 + blk * K
        bsafe = jnp.minimum(base, N_NODES - K)
        off = base - bsafe
        pltpu.sync_copy(x_hbm.at[pl.ds(bsafe, K), :], xbuf)
        pltpu.sync_copy(batch_hbm.at[pl.ds(bsafe, K)], ibuf.at[pl.ds(0, K)])

        def row_body(r, carry2):
            seg = ibuf[pl.ds(r, 16)][0]

            @pl.when((seg >= g0) & (seg < g1))
            def _():
                rel = seg - g0
                for c in range(NCH):
                    acc[rel, pl.ds(c * 16, 16)] += xbuf[r, pl.ds(c * 16, 16)]

            return carry2

        lax.fori_loop(off, K, row_body, 0)
        return carry

    lax.fori_loop(0, nblk, blk_body, 0)
    pltpu.sync_copy(acc, out_hbm.at[pl.ds(g0, SEG_PER_W), :])


def kernel(x, batch, W, b):
    del W, b
    probes = jnp.arange(0, G + 1, SEG_PER_W, dtype=jnp.int32)
    starts = jnp.searchsorted(batch, probes).astype(jnp.int32)
    starts = jnp.concatenate(
        [starts, jnp.full((15,), N_NODES, jnp.int32)])

    sc_kernel = pl.kernel(
        _pool_body,
        out_type=jax.ShapeDtypeStruct((G, C), jnp.float32),
        mesh=plsc.VectorSubcoreMesh(core_axis_name="c", subcore_axis_name="s"),
        scratch_types=[
            pltpu.VMEM((48,), jnp.int32),
            pltpu.VMEM((K + 16,), jnp.int32),
            pltpu.VMEM((K, C), jnp.float32),
            pltpu.VMEM((SEG_PER_W, C), jnp.float32),
        ],
    )
    return sc_kernel(x, batch, starts)


# speedup vs baseline: 4.0972x; 2.4104x over previous
import jax
import jax.numpy as jnp
from jax import lax
from jax.experimental import pallas as pl
from jax.experimental.pallas import tpu as pltpu
from jax.experimental.pallas import tpu_sc as plsc

N_NODES = 100000
C = 128
G = 1024
NC = 2
NS = 16
NW = NC * NS
SEG_PER_W = G // NW
K = 256
NCH = C // 16


def _pool_body(x_hbm, batch_hbm, starts_hbm, out_hbm,
               sbuf, ib0, ib1, xb0, xb1, acc, sx0, sx1, si0, si1):
    wid = lax.axis_index("s") * NC + lax.axis_index("c")
    g0 = wid * SEG_PER_W

    pltpu.sync_copy(starts_hbm, sbuf)
    bounds = sbuf[pl.ds(wid, 16)]
    lo = bounds[0]
    hi = bounds[1]
    base0 = (lo // 8) * 8
    nblk = (hi - base0 + (K - 1)) // K
    npair = (nblk + 1) // 2
    nblk_pad = npair * 2

    zero = jnp.zeros((16,), jnp.float32)
    for s in range(SEG_PER_W):
        for c in range(NCH):
            acc[s, pl.ds(c * 16, 16)] = zero

    xbufs = (xb0, xb1)
    ibufs = (ib0, ib1)
    xsems = (sx0, sx1)
    isems = (si0, si1)

    def dma_pair(blkid, b):
        base = base0 + blkid * K
        bsafe = jnp.minimum(base, N_NODES - K)
        xcp = pltpu.make_async_copy(
            x_hbm.at[pl.ds(bsafe, K), :], xbufs[b], xsems[b])
        icp = pltpu.make_async_copy(
            batch_hbm.at[pl.ds(bsafe, K)], ibufs[b].at[pl.ds(0, K)], isems[b])
        return xcp, icp

    def start_dma(blkid, b):
        xcp, icp = dma_pair(blkid, b)
        xcp.start()
        icp.start()

    @pl.when(nblk > 0)
    def _():
        start_dma(0, 0)

    def process_block(blkid, b, carry):
        base = base0 + blkid * K
        bsafe = jnp.minimum(base, N_NODES - K)
        xcp, icp = dma_pair(blkid, b)
        xcp.wait()
        icp.wait()

        @pl.when(blkid + 1 < nblk_pad)
        def _():
            start_dma(blkid + 1, 1 - b)

        rs = jnp.maximum(base, lo) - bsafe
        re = jnp.minimum(base + K, hi) - bsafe
        ib = ibufs[b]
        xb = xbufs[b]

        def row_body(r, rcarry):
            cur = rcarry[0]
            vs = rcarry[1:]
            seg = ib[pl.ds(r, 16)][0]
            changed = seg != cur
            new_vs = []
            for c in range(NCH):
                xv = xb[r, pl.ds(c * 16, 16)]
                new_vs.append(jnp.where(changed, zero, vs[c]) + xv)
            rel = seg - g0
            for c in range(NCH):
                acc[rel, pl.ds(c * 16, 16)] = new_vs[c]
            return (seg, *new_vs)

        return lax.fori_loop(rs, re, row_body, carry)

    def pair_body(i, carry):
        for b in range(2):
            carry = process_block(2 * i + b, b, carry)
        return carry

    init = (g0, *([zero] * NCH))
    lax.fori_loop(0, npair, pair_body, init)

    pltpu.sync_copy(acc, out_hbm.at[pl.ds(g0, SEG_PER_W), :])


def kernel(x, batch, W, b):
    del W, b
    probes = jnp.arange(0, G + 1, SEG_PER_W, dtype=jnp.int32)
    starts = jnp.searchsorted(batch, probes).astype(jnp.int32)
    starts = jnp.concatenate(
        [starts, jnp.full((15,), N_NODES, jnp.int32)])

    sc_kernel = pl.kernel(
        _pool_body,
        out_type=jax.ShapeDtypeStruct((G, C), jnp.float32),
        mesh=plsc.VectorSubcoreMesh(core_axis_name="c", subcore_axis_name="s"),
        scratch_types=[
            pltpu.VMEM((48,), jnp.int32),
            pltpu.VMEM((K + 16,), jnp.int32),
            pltpu.VMEM((K + 16,), jnp.int32),
            pltpu.VMEM((K, C), jnp.float32),
            pltpu.VMEM((K, C), jnp.float32),
            pltpu.VMEM((SEG_PER_W, C), jnp.float32),
            pltpu.SemaphoreType.DMA,
            pltpu.SemaphoreType.DMA,
            pltpu.SemaphoreType.DMA,
            pltpu.SemaphoreType.DMA,
        ],
    )
    return sc_kernel(x, batch, starts)


# speedup vs baseline: 5.6803x; 1.3864x over previous
import jax
import jax.numpy as jnp
from jax import lax
from jax.experimental import pallas as pl
from jax.experimental.pallas import tpu as pltpu
from jax.experimental.pallas import tpu_sc as plsc

N_NODES = 100000
C = 128
G = 1024
NC = 2
NS = 16
NW = NC * NS
SEG_PER_W = G // NW
K = 256
NCH = C // 16


def _pool_body(x_hbm, batch_hbm, starts_hbm, out_hbm,
               sbuf, ib0, ib1, xb0, xb1, acc, sx0, sx1, si0, si1):
    wid = lax.axis_index("s") * NC + lax.axis_index("c")
    g0 = wid * SEG_PER_W

    pltpu.sync_copy(starts_hbm, sbuf)
    bounds = sbuf[pl.ds(wid, 16)]
    lo = bounds[0]
    hi = bounds[1]
    base0 = (lo // 8) * 8
    nblk = (hi - base0 + (K - 1)) // K
    npair = (nblk + 1) // 2
    nblk_pad = npair * 2

    zero = jnp.zeros((16,), jnp.float32)
    for s in range(SEG_PER_W):
        for c in range(NCH):
            acc[s, pl.ds(c * 16, 16)] = zero

    xbufs = (xb0, xb1)
    ibufs = (ib0, ib1)
    xsems = (sx0, sx1)
    isems = (si0, si1)

    def dma_pair(blkid, b):
        base = base0 + blkid * K
        bsafe = jnp.minimum(base, N_NODES - K)
        xcp = pltpu.make_async_copy(
            x_hbm.at[pl.ds(bsafe, K), :], xbufs[b], xsems[b])
        icp = pltpu.make_async_copy(
            batch_hbm.at[pl.ds(bsafe, K)], ibufs[b].at[pl.ds(0, K)], isems[b])
        return xcp, icp

    def start_dma(blkid, b):
        xcp, icp = dma_pair(blkid, b)
        xcp.start()
        icp.start()

    @pl.when(nblk > 0)
    def _():
        start_dma(0, 0)

    def process_block(blkid, b, carry):
        base = base0 + blkid * K
        bsafe = jnp.minimum(base, N_NODES - K)
        xcp, icp = dma_pair(blkid, b)
        xcp.wait()
        icp.wait()

        @pl.when(blkid + 1 < nblk_pad)
        def _():
            start_dma(blkid + 1, 1 - b)

        rs = jnp.maximum(base, lo) - bsafe
        re = jnp.minimum(base + K, hi) - bsafe
        ib = ibufs[b]
        xb = xbufs[b]

        def one_row(r, seg, rcarry):
            cur = rcarry[0]
            vs = rcarry[1:]
            changed = seg != cur
            new_vs = []
            for c in range(NCH):
                xv = xb[r, pl.ds(c * 16, 16)]
                new_vs.append(jnp.where(changed, zero, vs[c]) + xv)
            rel = seg - g0
            for c in range(NCH):
                acc[rel, pl.ds(c * 16, 16)] = new_vs[c]
            return (seg, *new_vs)

        U = 16

        def group_body(t, rcarry):
            r0 = rs + t * U
            idxv = ib[pl.ds(r0, U)]
            for u in range(U):
                rcarry = one_row(r0 + u, idxv[u], rcarry)
            return rcarry

        ngrp = jnp.maximum(re - rs, 0) // U
        carry = lax.fori_loop(0, ngrp, group_body, carry)

        def tail_body(r, rcarry):
            return one_row(r, ib[pl.ds(r, 16)][0], rcarry)

        return lax.fori_loop(rs + ngrp * U, re, tail_body, carry)

    def pair_body(i, carry):
        for b in range(2):
            carry = process_block(2 * i + b, b, carry)
        return carry

    init = (g0, *([zero] * NCH))
    lax.fori_loop(0, npair, pair_body, init)

    pltpu.sync_copy(acc, out_hbm.at[pl.ds(g0, SEG_PER_W), :])


def kernel(x, batch, W, b):
    del W, b
    probes = jnp.arange(0, G + 1, SEG_PER_W, dtype=jnp.int32)
    starts = jnp.searchsorted(batch, probes).astype(jnp.int32)
    starts = jnp.concatenate(
        [starts, jnp.full((15,), N_NODES, jnp.int32)])

    sc_kernel = pl.kernel(
        _pool_body,
        out_type=jax.ShapeDtypeStruct((G, C), jnp.float32),
        mesh=plsc.VectorSubcoreMesh(core_axis_name="c", subcore_axis_name="s"),
        scratch_types=[
            pltpu.VMEM((48,), jnp.int32),
            pltpu.VMEM((K + 16,), jnp.int32),
            pltpu.VMEM((K + 16,), jnp.int32),
            pltpu.VMEM((K, C), jnp.float32),
            pltpu.VMEM((K, C), jnp.float32),
            pltpu.VMEM((SEG_PER_W, C), jnp.float32),
            pltpu.SemaphoreType.DMA,
            pltpu.SemaphoreType.DMA,
            pltpu.SemaphoreType.DMA,
            pltpu.SemaphoreType.DMA,
        ],
    )
    return sc_kernel(x, batch, starts)
